# Initial kernel scaffold; baseline (speedup 1.0000x reference)
import jax
import jax.numpy as jnp
from jax.experimental import pallas as pl


def kernel(track_name_ids, album_name_ids, artist_uri_ids, track_uri_ids, album_uri_ids, artist_name_ids, duration_ids, track_pop_ids, artist_pop_ids, artist_followers_ids, genres_ids, track_name_table, album_name_table, artist_uri_table, track_uri_table, album_uri_table, artist_name_table, duration_table, track_pop_table, artist_pop_table, artist_followers_table, genres_table, W0, b0, W1, b1, W2, b2):
    raise NotImplementedError("write your pallas kernel here")


# speedup vs baseline: 2.2565x; 2.2565x over previous
import functools

import jax
import jax.numpy as jnp
from jax import lax
from jax.experimental import pallas as pl
from jax.experimental.pallas import tpu as pltpu
from jax.experimental.pallas import tpu_sc as plsc

B = 1024
L = 16
EMB = 128
NW = 32
BPW = B // NW
NCHUNK = (BPW * L) // 128

_MESH = plsc.VectorSubcoreMesh(core_axis_name="c", subcore_axis_name="s")


def _sc_body(tn_ids, an_ids, ge_ids,
             au_ids, tu_ids, alu_ids, arn_ids, du_ids, tp_ids, ap_ids, af_ids,
             tn_tab, an_tab, ge_tab,
             au_tab, tu_tab, alu_tab, arn_tab, du_tab, tp_tab, ap_tab, af_tab,
             out,
             idx_seq, idx_one, dst_idx, rows_v, stage_v, zero_v, acc_sp, sem):
    c = lax.axis_index("c")
    s = lax.axis_index("s")
    wid = s * 2 + c
    base = wid * BPW
    accbase = s * BPW

    def _zero(i, _):
        zero_v[i // 8, pl.ds((i % 8) * 16, 16)] = jnp.zeros((16,), jnp.float32)
        return 0
    lax.fori_loop(0, BPW * 8, _zero, 0)

    for i in range(BPW):
        dst_idx[i // 8, pl.ds((i % 8) * 16, 16)] = (
            jnp.zeros((16,), jnp.int32) + (accbase + i))

    singles = [(au_ids, au_tab, 2), (tu_ids, tu_tab, 3), (alu_ids, alu_tab, 4),
               (arn_ids, arn_tab, 5), (du_ids, du_tab, 6), (tp_ids, tp_tab, 7),
               (ap_ids, ap_tab, 8), (af_ids, af_tab, 9)]
    for ids, tab, f in singles:
        pltpu.sync_copy(ids.at[pl.ds(base, BPW)], idx_one)
        pltpu.async_copy(tab.at[idx_one], stage_v, sem).wait()
        pltpu.sync_copy(stage_v, out.at[f, pl.ds(base, BPW)])

    seqs = [(tn_ids, tn_tab, 0), (an_ids, an_tab, 1), (ge_ids, ge_tab, 10)]
    for ids2d, tab, f in seqs:
        pltpu.sync_copy(ids2d.at[pl.ds(wid * NCHUNK, NCHUNK)], idx_seq)
        pltpu.sync_copy(zero_v, acc_sp.at[pl.ds(accbase, BPW)])
        handles = []
        for j in range(NCHUNK):
            handles.append(pltpu.async_copy(
                tab.at[idx_seq.at[j]], rows_v.at[pl.ds(j * 128, 128)], sem))
        for h in handles:
            h.wait()
        for j in range(NCHUNK):
            pltpu.sync_copy(rows_v.at[pl.ds(j * 128, 128)],
                            acc_sp.at[dst_idx.at[j]], add=True)
        pltpu.sync_copy(acc_sp.at[pl.ds(accbase, BPW)],
                        out.at[f, pl.ds(base, BPW)])


_sc_gather = functools.partial(
    pl.kernel,
    out_type=jax.ShapeDtypeStruct((11, B, EMB), jnp.float32),
    mesh=_MESH,
    scratch_types=[
        pltpu.VMEM((NCHUNK, 128), jnp.int32),
        pltpu.VMEM((BPW,), jnp.int32),
        pltpu.VMEM((NCHUNK, 128), jnp.int32),
        pltpu.VMEM((BPW * L, EMB), jnp.float32),
        pltpu.VMEM((BPW, EMB), jnp.float32),
        pltpu.VMEM((BPW, EMB), jnp.float32),
        pltpu.VMEM_SHARED((16 * BPW, EMB), jnp.float32),
        pltpu.SemaphoreType.DMA,
    ],
)(_sc_body)


def _tc_body(embs, tn_ids, ge_ids, r0tn, r0ge, w0, b0, w1, b1, w2, b2, out):
    def masked_fix(sums, ids, row0):
        nz = jnp.sum((ids[...] == 0).astype(jnp.float32), axis=1, keepdims=True)
        cnt = jnp.maximum(jnp.float32(L) - nz, 1.0)
        return (sums - nz * row0[...]) / cnt

    f0 = masked_fix(embs[0], tn_ids, r0tn)
    f1 = embs[1] * jnp.float32(1.0 / L)
    f10 = masked_fix(embs[10], ge_ids, r0ge)

    def mm(x, lo):
        return jax.lax.dot_general(
            x, w0[lo:lo + EMB, :], (((1,), (0,)), ((), ())),
            preferred_element_type=jnp.float32)

    acc = mm(f0, 0) + mm(f1, EMB) + mm(f10, 10 * EMB)
    for f in range(2, 10):
        acc = acc + mm(embs[f], f * EMB)
    h = jnp.maximum(acc + b0[...], 0.0)
    h = jnp.maximum(jax.lax.dot_general(h, w1[...], (((1,), (0,)), ((), ())),
                                        preferred_element_type=jnp.float32)
                    + b1[...], 0.0)
    h = jax.lax.dot_general(h, w2[...], (((1,), (0,)), ((), ())),
                            preferred_element_type=jnp.float32) + b2[...]
    ss = jnp.sum(h * h, axis=1, keepdims=True)
    out[...] = h / jnp.sqrt(jnp.maximum(ss, 1e-12))


def _tc_tower(embs, tn_ids, ge_ids, r0tn, r0ge, w0, b0, w1, b1, w2, b2):
    return pl.pallas_call(
        _tc_body,
        out_shape=jax.ShapeDtypeStruct((B, EMB), jnp.float32),
    )(embs, tn_ids, ge_ids, r0tn, r0ge, w0, b0, w1, b1, w2, b2)


def kernel(track_name_ids, album_name_ids, artist_uri_ids, track_uri_ids,
           album_uri_ids, artist_name_ids, duration_ids, track_pop_ids,
           artist_pop_ids, artist_followers_ids, genres_ids,
           track_name_table, album_name_table, artist_uri_table,
           track_uri_table, album_uri_table, artist_name_table,
           duration_table, track_pop_table, artist_pop_table,
           artist_followers_table, genres_table, W0, b0, W1, b1, W2, b2):
    i32 = lambda x: jnp.asarray(x, jnp.int32)
    tn2d = i32(track_name_ids).reshape(B * L // 128, 128)
    an2d = i32(album_name_ids).reshape(B * L // 128, 128)
    ge2d = i32(genres_ids).reshape(B * L // 128, 128)

    embs = _sc_gather(
        tn2d, an2d, ge2d,
        i32(artist_uri_ids), i32(track_uri_ids), i32(album_uri_ids),
        i32(artist_name_ids), i32(duration_ids), i32(track_pop_ids),
        i32(artist_pop_ids), i32(artist_followers_ids),
        track_name_table, album_name_table, genres_table,
        artist_uri_table, track_uri_table, album_uri_table,
        artist_name_table, duration_table, track_pop_table,
        artist_pop_table, artist_followers_table)

    return _tc_tower(
        embs, i32(track_name_ids), i32(genres_ids),
        track_name_table[0:1], genres_table[0:1],
        W0, b0.reshape(1, -1), W1, b1.reshape(1, -1), W2, b2.reshape(1, -1))


# speedup vs baseline: 3.0695x; 1.3603x over previous
import functools

import jax
import jax.numpy as jnp
from jax import lax
from jax.experimental import pallas as pl
from jax.experimental.pallas import tpu as pltpu
from jax.experimental.pallas import tpu_sc as plsc

B = 1024
L = 16
EMB = 128
NW = 32
BPW = B // NW
NCHUNK = (BPW * L) // 128

_MESH = plsc.VectorSubcoreMesh(core_axis_name="c", subcore_axis_name="s")


NSLOT = 4


def _sc_body(tn_ids, an_ids, ge_ids,
             au_ids, tu_ids, alu_ids, arn_ids, du_ids, tp_ids, ap_ids, af_ids,
             tn_tab, an_tab, ge_tab,
             au_tab, tu_tab, alu_tab, arn_tab, du_tab, tp_tab, ap_tab, af_tab,
             out,
             idx_seq, idx_sing, dst_idx, qbuf, sing_v, zero_v,
             acc0, acc1, acc2,
             sem_ids, sem_z, sem_sing, sem_out, sem_g, sem_a):
    c = lax.axis_index("c")
    s = lax.axis_index("s")
    wid = s * 2 + c
    base = wid * BPW
    accbase = s * BPW
    accs = [acc0, acc1, acc2]

    seqs = [(tn_ids, tn_tab, 0), (an_ids, an_tab, 1), (ge_ids, ge_tab, 10)]
    singles = [(au_ids, au_tab, 2), (tu_ids, tu_tab, 3), (alu_ids, alu_tab, 4),
               (arn_ids, arn_tab, 5), (du_ids, du_tab, 6), (tp_ids, tp_tab, 7),
               (ap_ids, ap_tab, 8), (af_ids, af_tab, 9)]
    h_ids = []
    for fi, (ids2d, _, _) in enumerate(seqs):
        h_ids.append(pltpu.async_copy(
            ids2d.at[pl.ds(wid * NCHUNK, NCHUNK)],
            idx_seq.at[pl.ds(fi * NCHUNK, NCHUNK)], sem_ids))
    for k, (ids, _, _) in enumerate(singles):
        h_ids.append(pltpu.async_copy(
            ids.at[pl.ds(base, BPW)], idx_sing.at[k], sem_ids))

    def _zero(i, _):
        zero_v[i // 8, pl.ds((i % 8) * 16, 16)] = jnp.zeros((16,), jnp.float32)
        return 0
    lax.fori_loop(0, BPW * 8, _zero, 0)
    for i in range(BPW):
        dst_idx[i // 8, pl.ds((i % 8) * 16, 16)] = (
            jnp.zeros((16,), jnp.int32) + (accbase + i))

    h_z = [pltpu.async_copy(zero_v, a.at[pl.ds(accbase, BPW)], sem_z)
           for a in accs]
    for h in h_ids:
        h.wait()

    h_sing = []
    for k, (_, tab, _) in enumerate(singles):
        h_sing.append(pltpu.async_copy(
            tab.at[idx_sing.at[k]], sing_v.at[pl.ds(k * BPW, BPW)], sem_sing))

    quarters = [(fi, j) for fi in range(3) for j in range(4)]
    NQ = len(quarters)

    def gather_q(qi, slot):
        fi, j = quarters[qi]
        tab = seqs[fi][1]
        return pltpu.async_copy(
            tab.at[idx_seq.at[fi * NCHUNK + j]],
            qbuf.at[slot], sem_g.at[slot])

    for h in h_z:
        h.wait()
    h_g = {}
    for qi in range(NSLOT):
        h_g[qi] = gather_q(qi, qi)

    h_a = {}
    h_out = []
    for qi in range(NQ):
        fi, j = quarters[qi]
        slot = qi % NSLOT
        h_g[qi].wait()
        h_a[qi] = pltpu.async_copy(
            qbuf.at[slot], accs[fi].at[dst_idx.at[j]], sem_a.at[slot], add=True)
        if qi + NSLOT < NQ:
            h_a[qi].wait()
            h_g[qi + NSLOT] = gather_q(qi + NSLOT, slot)
            if j == 3:
                h_out.append(pltpu.async_copy(
                    accs[fi].at[pl.ds(accbase, BPW)],
                    out.at[seqs[fi][2], pl.ds(base, BPW)], sem_out))

    for qi in range(NQ - NSLOT, NQ):
        h_a[qi].wait()
    h_out.append(pltpu.async_copy(
        accs[2].at[pl.ds(accbase, BPW)],
        out.at[seqs[2][2], pl.ds(base, BPW)], sem_out))

    for h in h_sing:
        h.wait()
    for k, (_, _, f) in enumerate(singles):
        h_out.append(pltpu.async_copy(
            sing_v.at[pl.ds(k * BPW, BPW)], out.at[f, pl.ds(base, BPW)],
            sem_out))
    for h in h_out:
        h.wait()


_sc_gather = functools.partial(
    pl.kernel,
    out_type=jax.ShapeDtypeStruct((11, B, EMB), jnp.float32),
    mesh=_MESH,
    scratch_types=[
        pltpu.VMEM((3 * NCHUNK, 128), jnp.int32),
        pltpu.VMEM((8, BPW), jnp.int32),
        pltpu.VMEM((NCHUNK, 128), jnp.int32),
        pltpu.VMEM((NSLOT, 128, EMB), jnp.float32),
        pltpu.VMEM((8 * BPW, EMB), jnp.float32),
        pltpu.VMEM((BPW, EMB), jnp.float32),
        pltpu.VMEM_SHARED((16 * BPW, EMB), jnp.float32),
        pltpu.VMEM_SHARED((16 * BPW, EMB), jnp.float32),
        pltpu.VMEM_SHARED((16 * BPW, EMB), jnp.float32),
        pltpu.SemaphoreType.DMA,
        pltpu.SemaphoreType.DMA,
        pltpu.SemaphoreType.DMA,
        pltpu.SemaphoreType.DMA,
        pltpu.SemaphoreType.DMA((NSLOT,)),
        pltpu.SemaphoreType.DMA((NSLOT,)),
    ],
)(_sc_body)


def _tc_body(embs, tn_ids, ge_ids, r0tn, r0ge, w0, b0, w1, b1, w2, b2, out):
    def masked_fix(sums, ids, row0):
        nz = jnp.sum((ids[...] == 0).astype(jnp.float32), axis=1, keepdims=True)
        cnt = jnp.maximum(jnp.float32(L) - nz, 1.0)
        return (sums - nz * row0[...]) / cnt

    f0 = masked_fix(embs[0], tn_ids, r0tn)
    f1 = embs[1] * jnp.float32(1.0 / L)
    f10 = masked_fix(embs[10], ge_ids, r0ge)

    def mm(x, lo):
        return jax.lax.dot_general(
            x, w0[lo:lo + EMB, :], (((1,), (0,)), ((), ())),
            preferred_element_type=jnp.float32)

    acc = mm(f0, 0) + mm(f1, EMB) + mm(f10, 10 * EMB)
    for f in range(2, 10):
        acc = acc + mm(embs[f], f * EMB)
    h = jnp.maximum(acc + b0[...], 0.0)
    h = jnp.maximum(jax.lax.dot_general(h, w1[...], (((1,), (0,)), ((), ())),
                                        preferred_element_type=jnp.float32)
                    + b1[...], 0.0)
    h = jax.lax.dot_general(h, w2[...], (((1,), (0,)), ((), ())),
                            preferred_element_type=jnp.float32) + b2[...]
    ss = jnp.sum(h * h, axis=1, keepdims=True)
    out[...] = h / jnp.sqrt(jnp.maximum(ss, 1e-12))


def _tc_tower(embs, tn_ids, ge_ids, r0tn, r0ge, w0, b0, w1, b1, w2, b2):
    return pl.pallas_call(
        _tc_body,
        out_shape=jax.ShapeDtypeStruct((B, EMB), jnp.float32),
    )(embs, tn_ids, ge_ids, r0tn, r0ge, w0, b0, w1, b1, w2, b2)


def kernel(track_name_ids, album_name_ids, artist_uri_ids, track_uri_ids,
           album_uri_ids, artist_name_ids, duration_ids, track_pop_ids,
           artist_pop_ids, artist_followers_ids, genres_ids,
           track_name_table, album_name_table, artist_uri_table,
           track_uri_table, album_uri_table, artist_name_table,
           duration_table, track_pop_table, artist_pop_table,
           artist_followers_table, genres_table, W0, b0, W1, b1, W2, b2):
    i32 = lambda x: jnp.asarray(x, jnp.int32)
    tn2d = i32(track_name_ids).reshape(B * L // 128, 128)
    an2d = i32(album_name_ids).reshape(B * L // 128, 128)
    ge2d = i32(genres_ids).reshape(B * L // 128, 128)

    embs = _sc_gather(
        tn2d, an2d, ge2d,
        i32(artist_uri_ids), i32(track_uri_ids), i32(album_uri_ids),
        i32(artist_name_ids), i32(duration_ids), i32(track_pop_ids),
        i32(artist_pop_ids), i32(artist_followers_ids),
        track_name_table, album_name_table, genres_table,
        artist_uri_table, track_uri_table, album_uri_table,
        artist_name_table, duration_table, track_pop_table,
        artist_pop_table, artist_followers_table)

    return _tc_tower(
        embs, i32(track_name_ids), i32(genres_ids),
        track_name_table[0:1], genres_table[0:1],
        W0, b0.reshape(1, -1), W1, b1.reshape(1, -1), W2, b2.reshape(1, -1))


# speedup vs baseline: 3.2225x; 1.0499x over previous
import functools

import jax
import jax.numpy as jnp
from jax import lax
from jax.experimental import pallas as pl
from jax.experimental.pallas import tpu as pltpu
from jax.experimental.pallas import tpu_sc as plsc

B = 1024
L = 16
EMB = 128
NW = 32
BPW = B // NW
NCHUNK = (BPW * L) // 128

_MESH = plsc.VectorSubcoreMesh(core_axis_name="c", subcore_axis_name="s")


NSLOT = 4


def _sc_body(tn_ids, an_ids, ge_ids,
             au_ids, tu_ids, alu_ids, arn_ids, du_ids, tp_ids, ap_ids, af_ids,
             tn_tab, an_tab, ge_tab,
             au_tab, tu_tab, alu_tab, arn_tab, du_tab, tp_tab, ap_tab, af_tab,
             out,
             idx_seq, idx_sing, qbuf, sing_v, ostage,
             sem_ids, sem_sing, sem_out, sem_g):
    c = lax.axis_index("c")
    s = lax.axis_index("s")
    wid = s * 2 + c
    base = wid * BPW

    seqs = [(tn_ids, tn_tab, 0), (an_ids, an_tab, 1), (ge_ids, ge_tab, 10)]
    singles = [(au_ids, au_tab, 2), (tu_ids, tu_tab, 3), (alu_ids, alu_tab, 4),
               (arn_ids, arn_tab, 5), (du_ids, du_tab, 6), (tp_ids, tp_tab, 7),
               (ap_ids, ap_tab, 8), (af_ids, af_tab, 9)]
    h_ids = []
    for fi, (ids2d, _, _) in enumerate(seqs):
        h_ids.append(pltpu.async_copy(
            ids2d.at[pl.ds(wid * NCHUNK, NCHUNK)],
            idx_seq.at[pl.ds(fi * NCHUNK, NCHUNK)], sem_ids))
    for k, (ids, _, _) in enumerate(singles):
        h_ids.append(pltpu.async_copy(
            ids.at[pl.ds(base, BPW)], idx_sing.at[k], sem_ids))
    for h in h_ids:
        h.wait()

    h_sing = []
    for k, (_, tab, _) in enumerate(singles):
        h_sing.append(pltpu.async_copy(
            tab.at[idx_sing.at[k]], sing_v.at[pl.ds(k * BPW, BPW)], sem_sing))

    quarters = [(fi, j) for fi in range(3) for j in range(4)]
    NQ = len(quarters)

    def gather_q(qi, slot):
        fi, j = quarters[qi]
        tab = seqs[fi][1]
        return pltpu.async_copy(
            tab.at[idx_seq.at[fi * NCHUNK + j]],
            qbuf.at[slot], sem_g.at[slot])

    h_g = {}
    for qi in range(NSLOT):
        h_g[qi] = gather_q(qi, qi)

    h_out = []
    for qi in range(NQ):
        fi, j = quarters[qi]
        slot = qi % NSLOT
        h_g[qi].wait()
        qb = qbuf.at[slot]
        ost = ostage.at[fi]

        def sum_elem(e, _):
            rb = e * L
            accs = [qb[rb, pl.ds(b * 16, 16)] for b in range(8)]
            for l in range(1, L):
                for b in range(8):
                    accs[b] = accs[b] + qb[rb + l, pl.ds(b * 16, 16)]
            for b in range(8):
                ost[j * 8 + e, pl.ds(b * 16, 16)] = accs[b]
            return 0
        lax.fori_loop(0, 8, sum_elem, 0)

        if qi + NSLOT < NQ:
            h_g[qi + NSLOT] = gather_q(qi + NSLOT, slot)
        if j == 3:
            h_out.append(pltpu.async_copy(
                ost, out.at[seqs[fi][2], pl.ds(base, BPW)], sem_out))

    for h in h_sing:
        h.wait()
    for k, (_, _, f) in enumerate(singles):
        h_out.append(pltpu.async_copy(
            sing_v.at[pl.ds(k * BPW, BPW)], out.at[f, pl.ds(base, BPW)],
            sem_out))
    for h in h_out:
        h.wait()


_sc_gather = functools.partial(
    pl.kernel,
    out_type=jax.ShapeDtypeStruct((11, B, EMB), jnp.float32),
    mesh=_MESH,
    scratch_types=[
        pltpu.VMEM((3 * NCHUNK, 128), jnp.int32),
        pltpu.VMEM((8, BPW), jnp.int32),
        pltpu.VMEM((NSLOT, 128, EMB), jnp.float32),
        pltpu.VMEM((8 * BPW, EMB), jnp.float32),
        pltpu.VMEM((3, BPW, EMB), jnp.float32),
        pltpu.SemaphoreType.DMA,
        pltpu.SemaphoreType.DMA,
        pltpu.SemaphoreType.DMA,
        pltpu.SemaphoreType.DMA((NSLOT,)),
    ],
)(_sc_body)


def _tc_body(embs, tn_ids, ge_ids, r0tn, r0ge, w0, b0, w1, b1, w2, b2, out):
    def masked_fix(sums, ids, row0):
        nz = jnp.sum((ids[...] == 0).astype(jnp.float32), axis=1, keepdims=True)
        cnt = jnp.maximum(jnp.float32(L) - nz, 1.0)
        return (sums - nz * row0[...]) / cnt

    f0 = masked_fix(embs[0], tn_ids, r0tn)
    f1 = embs[1] * jnp.float32(1.0 / L)
    f10 = masked_fix(embs[10], ge_ids, r0ge)

    def mm(x, lo):
        return jax.lax.dot_general(
            x, w0[lo:lo + EMB, :], (((1,), (0,)), ((), ())),
            preferred_element_type=jnp.float32)

    acc = mm(f0, 0) + mm(f1, EMB) + mm(f10, 10 * EMB)
    for f in range(2, 10):
        acc = acc + mm(embs[f], f * EMB)
    h = jnp.maximum(acc + b0[...], 0.0)
    h = jnp.maximum(jax.lax.dot_general(h, w1[...], (((1,), (0,)), ((), ())),
                                        preferred_element_type=jnp.float32)
                    + b1[...], 0.0)
    h = jax.lax.dot_general(h, w2[...], (((1,), (0,)), ((), ())),
                            preferred_element_type=jnp.float32) + b2[...]
    ss = jnp.sum(h * h, axis=1, keepdims=True)
    out[...] = h / jnp.sqrt(jnp.maximum(ss, 1e-12))


def _tc_tower(embs, tn_ids, ge_ids, r0tn, r0ge, w0, b0, w1, b1, w2, b2):
    return pl.pallas_call(
        _tc_body,
        out_shape=jax.ShapeDtypeStruct((B, EMB), jnp.float32),
    )(embs, tn_ids, ge_ids, r0tn, r0ge, w0, b0, w1, b1, w2, b2)


def kernel(track_name_ids, album_name_ids, artist_uri_ids, track_uri_ids,
           album_uri_ids, artist_name_ids, duration_ids, track_pop_ids,
           artist_pop_ids, artist_followers_ids, genres_ids,
           track_name_table, album_name_table, artist_uri_table,
           track_uri_table, album_uri_table, artist_name_table,
           duration_table, track_pop_table, artist_pop_table,
           artist_followers_table, genres_table, W0, b0, W1, b1, W2, b2):
    i32 = lambda x: jnp.asarray(x, jnp.int32)
    tn2d = i32(track_name_ids).reshape(B * L // 128, 128)
    an2d = i32(album_name_ids).reshape(B * L // 128, 128)
    ge2d = i32(genres_ids).reshape(B * L // 128, 128)

    embs = _sc_gather(
        tn2d, an2d, ge2d,
        i32(artist_uri_ids), i32(track_uri_ids), i32(album_uri_ids),
        i32(artist_name_ids), i32(duration_ids), i32(track_pop_ids),
        i32(artist_pop_ids), i32(artist_followers_ids),
        track_name_table, album_name_table, genres_table,
        artist_uri_table, track_uri_table, album_uri_table,
        artist_name_table, duration_table, track_pop_table,
        artist_pop_table, artist_followers_table)

    return _tc_tower(
        embs, i32(track_name_ids), i32(genres_ids),
        track_name_table[0:1], genres_table[0:1],
        W0, b0.reshape(1, -1), W1, b1.reshape(1, -1), W2, b2.reshape(1, -1))


# speedup vs baseline: 3.2375x; 1.0046x over previous
import functools

import jax
import jax.numpy as jnp
from jax import lax
from jax.experimental import pallas as pl
from jax.experimental.pallas import tpu as pltpu
from jax.experimental.pallas import tpu_sc as plsc

B = 1024
L = 16
EMB = 128
NW = 32
BPW = B // NW
NCHUNK = (BPW * L) // 128

_MESH = plsc.VectorSubcoreMesh(core_axis_name="c", subcore_axis_name="s")


NSLOT = 4


def _sc_body(tn_ids, an_ids, ge_ids,
             au_ids, tu_ids, alu_ids, arn_ids, du_ids, tp_ids, ap_ids, af_ids,
             tn_tab, an_tab, ge_tab,
             au_tab, tu_tab, alu_tab, arn_tab, du_tab, tp_tab, ap_tab, af_tab,
             out,
             idx_seq, idx_sing, qbuf, sing_v, ostage,
             sem_ids, sem_sing, sem_out, sem_g):
    c = lax.axis_index("c")
    s = lax.axis_index("s")
    wid = s * 2 + c
    base = wid * BPW

    seqs = [(tn_ids, tn_tab, 0), (an_ids, an_tab, 1), (ge_ids, ge_tab, 10)]
    singles = [(au_ids, au_tab, 2), (tu_ids, tu_tab, 3), (alu_ids, alu_tab, 4),
               (arn_ids, arn_tab, 5), (du_ids, du_tab, 6), (tp_ids, tp_tab, 7),
               (ap_ids, ap_tab, 8), (af_ids, af_tab, 9)]
    h_ids = []
    for fi, (ids2d, _, _) in enumerate(seqs):
        h_ids.append(pltpu.async_copy(
            ids2d.at[pl.ds(wid * NCHUNK, NCHUNK)],
            idx_seq.at[pl.ds(fi * NCHUNK, NCHUNK)], sem_ids))
    for k, (ids, _, _) in enumerate(singles):
        h_ids.append(pltpu.async_copy(
            ids.at[pl.ds(base, BPW)], idx_sing.at[k], sem_ids))
    for h in h_ids:
        h.wait()

    h_sing = []
    for k, (_, tab, _) in enumerate(singles):
        h_sing.append(pltpu.async_copy(
            tab.at[idx_sing.at[k]], sing_v.at[pl.ds(k * BPW, BPW)], sem_sing))

    quarters = [(fi, j) for fi in range(3) for j in range(4)]
    NQ = len(quarters)

    def gather_q(qi, slot):
        fi, j = quarters[qi]
        tab = seqs[fi][1]
        return pltpu.async_copy(
            tab.at[idx_seq.at[fi * NCHUNK + j]],
            qbuf.at[slot], sem_g.at[slot])

    h_g = {}
    for qi in range(NSLOT):
        h_g[qi] = gather_q(qi, qi)

    h_out = []
    for qi in range(NQ):
        fi, j = quarters[qi]
        slot = qi % NSLOT
        h_g[qi].wait()
        qb = qbuf.at[slot]
        ost = ostage.at[fi]

        def sum_elem(e, _):
            rb = e * L
            accs = [qb[rb, pl.ds(b * 16, 16)] for b in range(8)]
            for l in range(1, L):
                for b in range(8):
                    accs[b] = accs[b] + qb[rb + l, pl.ds(b * 16, 16)]
            for b in range(8):
                ost[j * 8 + e, pl.ds(b * 16, 16)] = accs[b]
            return 0
        lax.fori_loop(0, 8, sum_elem, 0)

        if qi + NSLOT < NQ:
            h_g[qi + NSLOT] = gather_q(qi + NSLOT, slot)
        if j == 3:
            h_out.append(pltpu.async_copy(
                ost, out.at[seqs[fi][2], pl.ds(base, BPW)], sem_out))

    for h in h_sing:
        h.wait()
    for k, (_, _, f) in enumerate(singles):
        h_out.append(pltpu.async_copy(
            sing_v.at[pl.ds(k * BPW, BPW)], out.at[f, pl.ds(base, BPW)],
            sem_out))
    for h in h_out:
        h.wait()


_sc_gather = functools.partial(
    pl.kernel,
    out_type=jax.ShapeDtypeStruct((11, B, EMB), jnp.float32),
    mesh=_MESH,
    scratch_types=[
        pltpu.VMEM((3 * NCHUNK, 128), jnp.int32),
        pltpu.VMEM((8, BPW), jnp.int32),
        pltpu.VMEM((NSLOT, 128, EMB), jnp.float32),
        pltpu.VMEM((8 * BPW, EMB), jnp.float32),
        pltpu.VMEM((3, BPW, EMB), jnp.float32),
        pltpu.SemaphoreType.DMA,
        pltpu.SemaphoreType.DMA,
        pltpu.SemaphoreType.DMA,
        pltpu.SemaphoreType.DMA((NSLOT,)),
    ],
)(_sc_body)


NB = 4
BB = B // NB


def _tc_body(embs, tn_ids, ge_ids, r0tn, r0ge, w0, b0, w1, b1, w2, b2, out):
    def masked_fix(sums, ids, row0):
        nz = jnp.sum((ids[...] == 0).astype(jnp.float32), axis=1, keepdims=True)
        cnt = jnp.maximum(jnp.float32(L) - nz, 1.0)
        return (sums - nz * row0[...]) / cnt

    f0 = masked_fix(embs[0], tn_ids, r0tn)
    f1 = embs[1] * jnp.float32(1.0 / L)
    f10 = masked_fix(embs[10], ge_ids, r0ge)

    def mm(x, lo):
        return jax.lax.dot_general(
            x, w0[lo:lo + EMB, :], (((1,), (0,)), ((), ())),
            preferred_element_type=jnp.float32)

    acc = mm(f0, 0) + mm(f1, EMB) + mm(f10, 10 * EMB)
    for f in range(2, 10):
        acc = acc + mm(embs[f], f * EMB)
    h = jnp.maximum(acc + b0[...], 0.0)
    h = jnp.maximum(jax.lax.dot_general(h, w1[...], (((1,), (0,)), ((), ())),
                                        preferred_element_type=jnp.float32)
                    + b1[...], 0.0)
    h = jax.lax.dot_general(h, w2[...], (((1,), (0,)), ((), ())),
                            preferred_element_type=jnp.float32) + b2[...]
    ss = jnp.sum(h * h, axis=1, keepdims=True)
    out[...] = h / jnp.sqrt(jnp.maximum(ss, 1e-12))


def _tc_tower(embs, tn_ids, ge_ids, r0tn, r0ge, w0, b0, w1, b1, w2, b2):
    full = lambda shape: pl.BlockSpec(shape, lambda b: (0,) * len(shape))
    return pl.pallas_call(
        _tc_body,
        grid=(NB,),
        in_specs=[
            pl.BlockSpec((11, BB, EMB), lambda b: (0, b, 0)),
            pl.BlockSpec((BB, L), lambda b: (b, 0)),
            pl.BlockSpec((BB, L), lambda b: (b, 0)),
            full((1, EMB)), full((1, EMB)),
            full((11 * EMB, 512)), full((1, 512)),
            full((512, 256)), full((1, 256)),
            full((256, EMB)), full((1, EMB)),
        ],
        out_specs=pl.BlockSpec((BB, EMB), lambda b: (b, 0)),
        out_shape=jax.ShapeDtypeStruct((B, EMB), jnp.float32),
    )(embs, tn_ids, ge_ids, r0tn, r0ge, w0, b0, w1, b1, w2, b2)


def kernel(track_name_ids, album_name_ids, artist_uri_ids, track_uri_ids,
           album_uri_ids, artist_name_ids, duration_ids, track_pop_ids,
           artist_pop_ids, artist_followers_ids, genres_ids,
           track_name_table, album_name_table, artist_uri_table,
           track_uri_table, album_uri_table, artist_name_table,
           duration_table, track_pop_table, artist_pop_table,
           artist_followers_table, genres_table, W0, b0, W1, b1, W2, b2):
    i32 = lambda x: jnp.asarray(x, jnp.int32)
    tn2d = i32(track_name_ids).reshape(B * L // 128, 128)
    an2d = i32(album_name_ids).reshape(B * L // 128, 128)
    ge2d = i32(genres_ids).reshape(B * L // 128, 128)

    embs = _sc_gather(
        tn2d, an2d, ge2d,
        i32(artist_uri_ids), i32(track_uri_ids), i32(album_uri_ids),
        i32(artist_name_ids), i32(duration_ids), i32(track_pop_ids),
        i32(artist_pop_ids), i32(artist_followers_ids),
        track_name_table, album_name_table, genres_table,
        artist_uri_table, track_uri_table, album_uri_table,
        artist_name_table, duration_table, track_pop_table,
        artist_pop_table, artist_followers_table)

    return _tc_tower(
        embs, i32(track_name_ids), i32(genres_ids),
        track_name_table[0:1], genres_table[0:1],
        W0, b0.reshape(1, -1), W1, b1.reshape(1, -1), W2, b2.reshape(1, -1))


# speedup vs baseline: 3.2595x; 1.0068x over previous
import functools

import jax
import jax.numpy as jnp
from jax import lax
from jax.experimental import pallas as pl
from jax.experimental.pallas import tpu as pltpu
from jax.experimental.pallas import tpu_sc as plsc

B = 1024
L = 16
EMB = 128
NW = 32
BPW = B // NW
NCHUNK = (BPW * L) // 128

_MESH = plsc.VectorSubcoreMesh(core_axis_name="c", subcore_axis_name="s")


NSLOT = 4


def _sc_body(tn_ids, an_ids, ge_ids,
             au_ids, tu_ids, alu_ids, arn_ids, du_ids, tp_ids, ap_ids, af_ids,
             tn_tab, an_tab, ge_tab,
             au_tab, tu_tab, alu_tab, arn_tab, du_tab, tp_tab, ap_tab, af_tab,
             out,
             idx_seq, idx_sing, qbuf, sing_v, ostage,
             sem_ids, sem_sing, sem_out, sem_g):
    c = lax.axis_index("c")
    s = lax.axis_index("s")
    wid = s * 2 + c
    base = wid * BPW

    seqs = [(tn_ids, tn_tab, 0), (an_ids, an_tab, 1), (ge_ids, ge_tab, 10)]
    singles = [(au_ids, au_tab, 2), (tu_ids, tu_tab, 3), (alu_ids, alu_tab, 4),
               (arn_ids, arn_tab, 5), (du_ids, du_tab, 6), (tp_ids, tp_tab, 7),
               (ap_ids, ap_tab, 8), (af_ids, af_tab, 9)]
    h_ids = []
    for fi, (ids2d, _, _) in enumerate(seqs):
        h_ids.append(pltpu.async_copy(
            ids2d.at[pl.ds(wid * NCHUNK, NCHUNK)],
            idx_seq.at[pl.ds(fi * NCHUNK, NCHUNK)], sem_ids))
    for k, (ids, _, _) in enumerate(singles):
        h_ids.append(pltpu.async_copy(
            ids.at[pl.ds(base, BPW)], idx_sing.at[k], sem_ids))
    for h in h_ids:
        h.wait()

    h_sing = []
    for k, (_, tab, _) in enumerate(singles):
        h_sing.append(pltpu.async_copy(
            tab.at[idx_sing.at[k]], sing_v.at[pl.ds(k * BPW, BPW)], sem_sing))

    quarters = [(fi, j) for fi in range(3) for j in range(4)]
    NQ = len(quarters)

    def gather_q(qi, slot):
        fi, j = quarters[qi]
        tab = seqs[fi][1]
        return pltpu.async_copy(
            tab.at[idx_seq.at[fi * NCHUNK + j]],
            qbuf.at[slot], sem_g.at[slot])

    h_g = {}
    for qi in range(NSLOT):
        h_g[qi] = gather_q(qi, qi)

    h_out = []
    for qi in range(NQ):
        fi, j = quarters[qi]
        slot = qi % NSLOT
        h_g[qi].wait()
        qb = qbuf.at[slot]
        ost = ostage.at[fi]

        def sum_elem(e, _):
            rb = e * L
            accs = [qb[rb, pl.ds(b * 16, 16)] for b in range(8)]
            for l in range(1, L):
                for b in range(8):
                    accs[b] = accs[b] + qb[rb + l, pl.ds(b * 16, 16)]
            for b in range(8):
                ost[j * 8 + e, pl.ds(b * 16, 16)] = accs[b]
            return 0
        lax.fori_loop(0, 8, sum_elem, 0)

        if qi + NSLOT < NQ:
            h_g[qi + NSLOT] = gather_q(qi + NSLOT, slot)
        if j == 3:
            h_out.append(pltpu.async_copy(
                ost, out.at[seqs[fi][2], pl.ds(base, BPW)], sem_out))

    for h in h_sing:
        h.wait()
    for k, (_, _, f) in enumerate(singles):
        h_out.append(pltpu.async_copy(
            sing_v.at[pl.ds(k * BPW, BPW)], out.at[f, pl.ds(base, BPW)],
            sem_out))
    for h in h_out:
        h.wait()


_sc_gather = functools.partial(
    pl.kernel,
    out_type=jax.ShapeDtypeStruct((11, B, EMB), jnp.float32),
    mesh=_MESH,
    scratch_types=[
        pltpu.VMEM((3 * NCHUNK, 128), jnp.int32),
        pltpu.VMEM((8, BPW), jnp.int32),
        pltpu.VMEM((NSLOT, 128, EMB), jnp.float32),
        pltpu.VMEM((8 * BPW, EMB), jnp.float32),
        pltpu.VMEM((3, BPW, EMB), jnp.float32),
        pltpu.SemaphoreType.DMA,
        pltpu.SemaphoreType.DMA,
        pltpu.SemaphoreType.DMA,
        pltpu.SemaphoreType.DMA((NSLOT,)),
    ],
)(_sc_body)


NB = 4
BB = B // NB


def _tc_body(embs, tn_ids, ge_ids, r0tn, r0ge, w0, b0, w1, b1, w2, b2, out):
    def masked_fix(sums, ids, row0):
        nz = jnp.sum((ids[...] == 0).astype(jnp.float32), axis=1, keepdims=True)
        cnt = jnp.maximum(jnp.float32(L) - nz, 1.0)
        return (sums - nz * row0[...]) / cnt

    f0 = masked_fix(embs[0], tn_ids, r0tn)
    f1 = embs[1] * jnp.float32(1.0 / L)
    f10 = masked_fix(embs[10], ge_ids, r0ge)

    def mm(x, lo):
        return jax.lax.dot_general(
            x.astype(jnp.bfloat16), w0[lo:lo + EMB, :],
            (((1,), (0,)), ((), ())), preferred_element_type=jnp.float32)

    acc = mm(f0, 0) + mm(f1, EMB) + mm(f10, 10 * EMB)
    for f in range(2, 10):
        acc = acc + mm(embs[f], f * EMB)
    h = jnp.maximum(acc + b0[...], 0.0)
    h = jnp.maximum(jax.lax.dot_general(h, w1[...], (((1,), (0,)), ((), ())),
                                        preferred_element_type=jnp.float32)
                    + b1[...], 0.0)
    h = jax.lax.dot_general(h, w2[...], (((1,), (0,)), ((), ())),
                            preferred_element_type=jnp.float32) + b2[...]
    ss = jnp.sum(h * h, axis=1, keepdims=True)
    out[...] = h / jnp.sqrt(jnp.maximum(ss, 1e-12))


def _tc_tower(embs, tn_ids, ge_ids, r0tn, r0ge, w0, b0, w1, b1, w2, b2):
    full = lambda shape: pl.BlockSpec(shape, lambda b: (0,) * len(shape))
    return pl.pallas_call(
        _tc_body,
        grid=(NB,),
        in_specs=[
            pl.BlockSpec((11, BB, EMB), lambda b: (0, b, 0)),
            pl.BlockSpec((BB, L), lambda b: (b, 0)),
            pl.BlockSpec((BB, L), lambda b: (b, 0)),
            full((1, EMB)), full((1, EMB)),
            full((11 * EMB, 512)), full((1, 512)),
            full((512, 256)), full((1, 256)),
            full((256, EMB)), full((1, EMB)),
        ],
        out_specs=pl.BlockSpec((BB, EMB), lambda b: (b, 0)),
        out_shape=jax.ShapeDtypeStruct((B, EMB), jnp.float32),
    )(embs, tn_ids, ge_ids, r0tn, r0ge, w0, b0, w1, b1, w2, b2)


def kernel(track_name_ids, album_name_ids, artist_uri_ids, track_uri_ids,
           album_uri_ids, artist_name_ids, duration_ids, track_pop_ids,
           artist_pop_ids, artist_followers_ids, genres_ids,
           track_name_table, album_name_table, artist_uri_table,
           track_uri_table, album_uri_table, artist_name_table,
           duration_table, track_pop_table, artist_pop_table,
           artist_followers_table, genres_table, W0, b0, W1, b1, W2, b2):
    i32 = lambda x: jnp.asarray(x, jnp.int32)
    tn2d = i32(track_name_ids).reshape(B * L // 128, 128)
    an2d = i32(album_name_ids).reshape(B * L // 128, 128)
    ge2d = i32(genres_ids).reshape(B * L // 128, 128)

    embs = _sc_gather(
        tn2d, an2d, ge2d,
        i32(artist_uri_ids), i32(track_uri_ids), i32(album_uri_ids),
        i32(artist_name_ids), i32(duration_ids), i32(track_pop_ids),
        i32(artist_pop_ids), i32(artist_followers_ids),
        track_name_table, album_name_table, genres_table,
        artist_uri_table, track_uri_table, album_uri_table,
        artist_name_table, duration_table, track_pop_table,
        artist_pop_table, artist_followers_table)

    return _tc_tower(
        embs, i32(track_name_ids), i32(genres_ids),
        track_name_table[0:1], genres_table[0:1],
        W0.astype(jnp.bfloat16), b0.reshape(1, -1), W1, b1.reshape(1, -1),
        W2, b2.reshape(1, -1))
